# baseline (speedup 1.0000x reference)
import functools

import jax
import jax.numpy as jnp
from jax import lax
from jax.experimental import pallas as pl
from jax.experimental.pallas import tpu as pltpu
from jax.experimental.pallas import tpu_sc as plsc

B = 4
N = 4096
NB = 4
RB = N // NB


def _dist_tile(src_ref, tgt_ref):
    a = src_ref[0]
    tt = tgt_ref[0]
    m2ab = jnp.dot(a, tt, preferred_element_type=jnp.float32)
    a2 = a[:, 3:4]
    b2 = tt[4:5, :]
    return (a2 + b2) + m2ab


def _fine_kernel(fine_ref, tgt_ref, rm_ref, am_ref, cm_ref):
    i = pl.program_id(1)
    d = _dist_tile(fine_ref, tgt_ref)
    rm = jnp.maximum(jnp.min(d, axis=1), 0.0)
    rm_ref[0, 0, :] = rm
    iota = lax.broadcasted_iota(jnp.int32, d.shape, 1)
    am_ref[0, 0, :] = jnp.min(jnp.where(d <= rm[:, None], iota, N), axis=1)
    cm = jnp.min(d, axis=0, keepdims=True)
    cm_ref[0] = jnp.where(i == 0, cm, jnp.minimum(cm_ref[0], cm))


def _stage_fine(fine_aug, tgt_aug):
    row_spec = pl.BlockSpec((1, 1, RB), lambda b, i: (b * NB + i, 0, 0))
    col_spec = pl.BlockSpec((1, 1, N), lambda b, i: (b, 0, 0))
    return pl.pallas_call(
        _fine_kernel,
        grid=(B, NB),
        in_specs=[
            pl.BlockSpec((1, RB, 8), lambda b, i: (b, i, 0)),
            pl.BlockSpec((1, 8, N), lambda b, i: (b, 0, 0)),
        ],
        out_specs=[row_spec, row_spec, col_spec],
        out_shape=[
            jax.ShapeDtypeStruct((B * NB, 1, RB), jnp.float32),
            jax.ShapeDtypeStruct((B * NB, 1, RB), jnp.int32),
            jax.ShapeDtypeStruct((B, 1, N), jnp.float32),
        ],
    )(fine_aug, tgt_aug)


def _coarse_kernel(coarse_ref, tgt_ref, rm_ref, cm_ref):
    i = pl.program_id(1)
    d = _dist_tile(coarse_ref, tgt_ref)
    rm_ref[0, 0, :] = jnp.min(d, axis=1)
    cm = jnp.min(d, axis=0, keepdims=True)
    cm_ref[0] = jnp.where(i == 0, cm, jnp.minimum(cm_ref[0], cm))


def _stage_coarse(coarse_aug, tgt_aug):
    row_spec = pl.BlockSpec((1, 1, RB), lambda b, i: (b * NB + i, 0, 0))
    col_spec = pl.BlockSpec((1, 1, N), lambda b, i: (b, 0, 0))
    return pl.pallas_call(
        _coarse_kernel,
        grid=(B, NB),
        in_specs=[
            pl.BlockSpec((1, RB, 8), lambda b, i: (b, i, 0)),
            pl.BlockSpec((1, 8, N), lambda b, i: (b, 0, 0)),
        ],
        out_specs=[row_spec, col_spec],
        out_shape=[
            jax.ShapeDtypeStruct((B * NB, 1, RB), jnp.float32),
            jax.ShapeDtypeStruct((B, 1, N), jnp.float32),
        ],
    )(coarse_aug, tgt_aug)


def _sc_gather(idx_flat, table_flat):
    info = plsc.get_sparse_core_info()
    nw = info.num_cores * info.num_subcores
    bpw = (B * N) // nw
    ch = 128
    mesh = plsc.VectorSubcoreMesh(core_axis_name="c", subcore_axis_name="s")

    @functools.partial(
        pl.kernel,
        out_type=jax.ShapeDtypeStruct((3 * B, N), jnp.float32),
        mesh=mesh,
        scratch_types=[
            pltpu.VMEM((bpw,), jnp.int32),
            pltpu.VMEM((3, bpw), jnp.int32),
            pltpu.VMEM((3, bpw), jnp.float32),
            pltpu.SemaphoreType.DMA,
        ],
        compiler_params=pltpu.CompilerParams(use_tc_tiling_on_sc=False),
    )
    def k(idx_hbm, table_hbm, out_hbm, idx_v, idx2_v, gat_v, sem):
        wid = lax.axis_index("s") * info.num_cores + lax.axis_index("c")
        base = wid * bpw
        b = base // N
        col0 = base - b * N
        pltpu.sync_copy(idx_hbm.at[pl.ds(base, bpw)], idx_v)
        for c in range(3):
            off = (b * 8 + c) * N
            for j in range(bpw // 16):
                sl = pl.ds(j * 16, 16)
                idx2_v[c, sl] = idx_v[sl] + off
        copies = [
            pltpu.async_copy(table_hbm.at[idx2_v.at[c, pl.ds(q * ch, ch)]],
                             gat_v.at[c, pl.ds(q * ch, ch)], sem)
            for c in range(3) for q in range(bpw // ch)
        ]
        for cp in copies:
            cp.wait()
        for c in range(3):
            pltpu.sync_copy(gat_v.at[c], out_hbm.at[b * 3 + c, pl.ds(col0, bpw)])

    return k(idx_flat, table_flat)


def _roll1(v, k):
    return jnp.concatenate([v[:, k:], v[:, :k]], axis=1)


def _wedge_volume(px, py, pz):
    x1, y1, z1 = _roll1(px, 1), _roll1(py, 1), _roll1(pz, 1)
    x2, y2, z2 = _roll1(px, 2), _roll1(py, 2), _roll1(pz, 2)
    cx = y1 * z2 - z1 * y2
    cy = z1 * x2 - x1 * z2
    cz = x1 * y2 - y1 * x2
    return jnp.sum(px * cx + py * cy + pz * cz) / 6.0


def _loss_kernel(f2_ref, n2_ref, frm_ref, fcm_ref, crm_ref, ccm_ref, out_ref):
    inv_bn = 1.0 / (B * N)

    def chamfer_halfsum(ref):
        return jnp.sum(jnp.sqrt(jnp.maximum(ref[...], 0.0)))

    la_f = 0.5 * (chamfer_halfsum(frm_ref) + chamfer_halfsum(fcm_ref)) * inv_bn
    la_c = 0.5 * (chamfer_halfsum(crm_ref) + chamfer_halfsum(ccm_ref)) * inv_bn
    f2 = f2_ref[...]
    n2 = n2_ref[...]
    loss_rot = 0.0
    loss_ref = 0.0
    loss_geo = 0.0
    for b in range(B):
        fx, fy, fz = (f2[3 * b:3 * b + 1], f2[3 * b + 1:3 * b + 2],
                      f2[3 * b + 2:3 * b + 3])
        nx, ny, nz = (n2[3 * b:3 * b + 1], n2[3 * b + 1:3 * b + 2],
                      n2[3 * b + 2:3 * b + 3])
        def prj(v):
            return v.astype(jnp.bfloat16).astype(jnp.float32)
        im = jnp.sqrt(jnp.sum(prj(fz) * prj(fz)))
        rm = jnp.sqrt(jnp.sum(prj(nz) * prj(nz)))
        loss_rot = loss_rot + (im - rm) ** 2
        loss_ref = loss_ref + jnp.sum((prj(fy) - prj(ny)) ** 2)
        vol_f = _wedge_volume(fx, fy, fz)
        vol_n = _wedge_volume(nx, ny, nz)
        loss_geo = loss_geo + (vol_f - vol_n) ** 2
    total = (loss_rot / B + loss_ref * inv_bn + la_f + la_c + loss_geo / B)
    out_ref[...] = jnp.reshape(total, (1, 1))


def _stage3(f2, n2, frm, fcm, crm, ccm):
    return pl.pallas_call(
        _loss_kernel,
        out_shape=jax.ShapeDtypeStruct((1, 1), jnp.float32),
    )(f2, n2, frm, fcm, crm, ccm)


def _augment_src(p):
    a2 = jnp.sum(p * p, axis=-1, keepdims=True)
    zero = jnp.zeros(p.shape[:-1] + (4,), jnp.float32)
    return jnp.concatenate([-2.0 * p, a2, zero], axis=-1)


def _augment_tgt(t):
    tt = jnp.swapaxes(t, 1, 2)
    b2 = jnp.sum(tt * tt, axis=1, keepdims=True)
    zero1 = jnp.zeros((B, 1, N), jnp.float32)
    zero3 = jnp.zeros((B, 3, N), jnp.float32)
    return jnp.concatenate([tt, zero1, b2, zero3], axis=1)


def kernel(source_points, target_points):
    fine = source_points[1]
    src_aug = _augment_src(source_points)
    coarse_aug = src_aug[0]
    fine_aug = src_aug[1]
    tgt_aug = _augment_tgt(target_points)

    frm, fam, fcm = _stage_fine(fine_aug, tgt_aug)
    n2 = _sc_gather(fam.reshape(B * N), tgt_aug.reshape(B * 8 * N))
    crm, ccm = _stage_coarse(coarse_aug, tgt_aug)

    f2 = jnp.swapaxes(fine, 1, 2).reshape(3 * B, N)
    out = _stage3(f2, n2, frm.reshape(B, N), fcm.reshape(B, N),
                  crm.reshape(B, N), ccm.reshape(B, N))
    return out[0, 0]


# speedup vs baseline: 1.0560x; 1.0560x over previous
import functools

import jax
import jax.numpy as jnp
from jax import lax
from jax.experimental import pallas as pl
from jax.experimental.pallas import tpu as pltpu
from jax.experimental.pallas import tpu_sc as plsc

B = 4
N = 4096
NB = 4
RB = N // NB


def _dist_tile(src_ref, tgt_ref):
    a = src_ref[0]
    tt = tgt_ref[0]
    m2ab = jnp.dot(a, tt, preferred_element_type=jnp.float32)
    a2 = a[:, 3:4]
    b2 = tt[4:5, :]
    return (a2 + b2) + m2ab


def _fine_kernel(fine_ref, tgt_ref, rm_ref, am_ref, cm_ref):
    i = pl.program_id(1)
    d = _dist_tile(fine_ref, tgt_ref)
    rm = jnp.maximum(jnp.min(d, axis=1), 0.0)
    rm_ref[0, 0, :] = rm
    iota = lax.broadcasted_iota(jnp.int32, (1, N), 1).astype(jnp.float32)
    amf = jnp.min(jnp.where(d <= rm[:, None], iota, float(N)), axis=1)
    am_ref[0, 0, :] = amf.astype(jnp.int32)
    cm = jnp.min(d, axis=0, keepdims=True)
    cm_ref[0] = jnp.where(i == 0, cm, jnp.minimum(cm_ref[0], cm))


def _stage_fine(fine_aug, tgt_aug):
    row_spec = pl.BlockSpec((1, 1, RB), lambda b, i: (b * NB + i, 0, 0))
    col_spec = pl.BlockSpec((1, 1, N), lambda b, i: (b, 0, 0))
    return pl.pallas_call(
        _fine_kernel,
        grid=(B, NB),
        in_specs=[
            pl.BlockSpec((1, RB, 8), lambda b, i: (b, i, 0)),
            pl.BlockSpec((1, 8, N), lambda b, i: (b, 0, 0)),
        ],
        out_specs=[row_spec, row_spec, col_spec],
        out_shape=[
            jax.ShapeDtypeStruct((B * NB, 1, RB), jnp.float32),
            jax.ShapeDtypeStruct((B * NB, 1, RB), jnp.int32),
            jax.ShapeDtypeStruct((B, 1, N), jnp.float32),
        ],
    )(fine_aug, tgt_aug)


def _coarse_kernel(coarse_ref, tgt_ref, rm_ref, cm_ref):
    i = pl.program_id(1)
    d = _dist_tile(coarse_ref, tgt_ref)
    rm_ref[0, 0, :] = jnp.min(d, axis=1)
    cm = jnp.min(d, axis=0, keepdims=True)
    cm_ref[0] = jnp.where(i == 0, cm, jnp.minimum(cm_ref[0], cm))


def _stage_coarse(coarse_aug, tgt_aug):
    row_spec = pl.BlockSpec((1, 1, RB), lambda b, i: (b * NB + i, 0, 0))
    col_spec = pl.BlockSpec((1, 1, N), lambda b, i: (b, 0, 0))
    return pl.pallas_call(
        _coarse_kernel,
        grid=(B, NB),
        in_specs=[
            pl.BlockSpec((1, RB, 8), lambda b, i: (b, i, 0)),
            pl.BlockSpec((1, 8, N), lambda b, i: (b, 0, 0)),
        ],
        out_specs=[row_spec, col_spec],
        out_shape=[
            jax.ShapeDtypeStruct((B * NB, 1, RB), jnp.float32),
            jax.ShapeDtypeStruct((B, 1, N), jnp.float32),
        ],
    )(coarse_aug, tgt_aug)


def _sc_gather(idx_flat, table_flat):
    info = plsc.get_sparse_core_info()
    nw = info.num_cores * info.num_subcores
    bpw = (B * N) // nw
    ch = 128
    mesh = plsc.VectorSubcoreMesh(core_axis_name="c", subcore_axis_name="s")

    @functools.partial(
        pl.kernel,
        out_type=jax.ShapeDtypeStruct((3 * B, N), jnp.float32),
        mesh=mesh,
        scratch_types=[
            pltpu.VMEM((bpw,), jnp.int32),
            pltpu.VMEM((3, bpw), jnp.int32),
            pltpu.VMEM((3, bpw), jnp.float32),
            pltpu.SemaphoreType.DMA,
        ],
        compiler_params=pltpu.CompilerParams(use_tc_tiling_on_sc=False),
    )
    def k(idx_hbm, table_hbm, out_hbm, idx_v, idx2_v, gat_v, sem):
        wid = lax.axis_index("s") * info.num_cores + lax.axis_index("c")
        base = wid * bpw
        b = base // N
        col0 = base - b * N
        pltpu.sync_copy(idx_hbm.at[pl.ds(base, bpw)], idx_v)
        for c in range(3):
            off = (b * 8 + c) * N
            for j in range(bpw // 16):
                sl = pl.ds(j * 16, 16)
                idx2_v[c, sl] = idx_v[sl] + off
        copies = [
            pltpu.async_copy(table_hbm.at[idx2_v.at[c, pl.ds(q * ch, ch)]],
                             gat_v.at[c, pl.ds(q * ch, ch)], sem)
            for c in range(3) for q in range(bpw // ch)
        ]
        for cp in copies:
            cp.wait()
        for c in range(3):
            pltpu.sync_copy(gat_v.at[c], out_hbm.at[b * 3 + c, pl.ds(col0, bpw)])

    return k(idx_flat, table_flat)


def _roll1(v, k):
    return jnp.concatenate([v[:, k:], v[:, :k]], axis=1)


def _wedge_volume(px, py, pz):
    x1, y1, z1 = _roll1(px, 1), _roll1(py, 1), _roll1(pz, 1)
    x2, y2, z2 = _roll1(px, 2), _roll1(py, 2), _roll1(pz, 2)
    cx = y1 * z2 - z1 * y2
    cy = z1 * x2 - x1 * z2
    cz = x1 * y2 - y1 * x2
    return jnp.sum(px * cx + py * cy + pz * cz) / 6.0


def _loss_kernel(f2_ref, n2_ref, frm_ref, fcm_ref, crm_ref, ccm_ref, out_ref):
    inv_bn = 1.0 / (B * N)

    def chamfer_halfsum(ref):
        return jnp.sum(jnp.sqrt(jnp.maximum(ref[...], 0.0)))

    la_f = 0.5 * (chamfer_halfsum(frm_ref) + chamfer_halfsum(fcm_ref)) * inv_bn
    la_c = 0.5 * (chamfer_halfsum(crm_ref) + chamfer_halfsum(ccm_ref)) * inv_bn
    f2 = f2_ref[...]
    n2 = n2_ref[...]
    loss_rot = 0.0
    loss_ref = 0.0
    loss_geo = 0.0
    for b in range(B):
        fx, fy, fz = (f2[3 * b:3 * b + 1], f2[3 * b + 1:3 * b + 2],
                      f2[3 * b + 2:3 * b + 3])
        nx, ny, nz = (n2[3 * b:3 * b + 1], n2[3 * b + 1:3 * b + 2],
                      n2[3 * b + 2:3 * b + 3])
        def prj(v):
            return v.astype(jnp.bfloat16).astype(jnp.float32)
        im = jnp.sqrt(jnp.sum(prj(fz) * prj(fz)))
        rm = jnp.sqrt(jnp.sum(prj(nz) * prj(nz)))
        loss_rot = loss_rot + (im - rm) ** 2
        loss_ref = loss_ref + jnp.sum((prj(fy) - prj(ny)) ** 2)
        vol_f = _wedge_volume(fx, fy, fz)
        vol_n = _wedge_volume(nx, ny, nz)
        loss_geo = loss_geo + (vol_f - vol_n) ** 2
    total = (loss_rot / B + loss_ref * inv_bn + la_f + la_c + loss_geo / B)
    out_ref[...] = jnp.reshape(total, (1, 1))


def _stage3(f2, n2, frm, fcm, crm, ccm):
    return pl.pallas_call(
        _loss_kernel,
        out_shape=jax.ShapeDtypeStruct((1, 1), jnp.float32),
    )(f2, n2, frm, fcm, crm, ccm)


def _augment_src(p):
    a2 = jnp.sum(p * p, axis=-1, keepdims=True)
    zero = jnp.zeros(p.shape[:-1] + (4,), jnp.float32)
    return jnp.concatenate([-2.0 * p, a2, zero], axis=-1)


def _augment_tgt(t):
    tt = jnp.swapaxes(t, 1, 2)
    b2 = jnp.sum(tt * tt, axis=1, keepdims=True)
    zero1 = jnp.zeros((B, 1, N), jnp.float32)
    zero3 = jnp.zeros((B, 3, N), jnp.float32)
    return jnp.concatenate([tt, zero1, b2, zero3], axis=1)


def kernel(source_points, target_points):
    fine = source_points[1]
    src_aug = _augment_src(source_points)
    coarse_aug = src_aug[0]
    fine_aug = src_aug[1]
    tgt_aug = _augment_tgt(target_points)

    frm, fam, fcm = _stage_fine(fine_aug, tgt_aug)
    n2 = _sc_gather(fam.reshape(B * N), tgt_aug.reshape(B * 8 * N))
    crm, ccm = _stage_coarse(coarse_aug, tgt_aug)

    f2 = jnp.swapaxes(fine, 1, 2).reshape(3 * B, N)
    out = _stage3(f2, n2, frm.reshape(B, N), fcm.reshape(B, N),
                  crm.reshape(B, N), ccm.reshape(B, N))
    return out[0, 0]
